# Initial kernel scaffold; baseline (speedup 1.0000x reference)
import jax
import jax.numpy as jnp
from jax.experimental import pallas as pl


def kernel(x, edge_index, W1, b1, W2, b2, W3, b3, W4, b4):
    raise NotImplementedError("write your pallas kernel here")


# speedup vs baseline: 2.6815x; 2.6815x over previous
import functools

import jax
import jax.numpy as jnp
from jax import lax
from jax.experimental import pallas as pl
from jax.experimental.pallas import tpu as pltpu
from jax.experimental.pallas import tpu_sc as plsc

N = 10000
D = 128
E = 320000
HID = 128

NC = 2
NS = 16
NW = NC * NS
BPW = 128
NBLK = 80
EPW = NBLK * BPW
EPAD = NW * EPW
RPS = 632
NP = NS * RPS


def _sc_body(x_hbm, row_hbm, col_hbm, zeros_hbm, out_hbm,
             idxr_v, idxc_v, rows_v, agg_sh, sem):
  c = lax.axis_index("c")
  s = lax.axis_index("s")
  w = s * NC + c
  base = w * EPW
  zbase = s * RPS

  pltpu.sync_copy(zeros_hbm.at[pl.ds(zbase, RPS)],
                  agg_sh.at[pl.ds(zbase, RPS)])
  plsc.subcore_barrier()

  @pl.loop(0, NBLK)
  def _blocks(j):
    off = base + j * BPW
    pltpu.sync_copy(row_hbm.at[pl.ds(off, BPW)], idxr_v)
    pltpu.sync_copy(col_hbm.at[pl.ds(off, BPW)], idxc_v)
    pltpu.async_copy(x_hbm.at[idxc_v], rows_v, sem).wait()
    pltpu.sync_copy(rows_v, agg_sh.at[idxr_v], add=True)

  plsc.subcore_barrier()
  pltpu.sync_copy(agg_sh.at[pl.ds(zbase, RPS)],
                  out_hbm.at[pl.ds(c * NP + zbase, RPS)])


_sc_agg = functools.partial(
    pl.kernel,
    out_type=jax.ShapeDtypeStruct((NC * NP, D), jnp.float32),
    mesh=plsc.VectorSubcoreMesh(core_axis_name="c", subcore_axis_name="s"),
    scratch_types=[
        pltpu.VMEM((BPW,), jnp.int32),
        pltpu.VMEM((BPW,), jnp.int32),
        pltpu.VMEM((BPW, D), jnp.float32),
        pltpu.VMEM_SHARED((NP, D), jnp.float32),
        pltpu.SemaphoreType.DMA,
    ],
)(_sc_body)


@jax.jit
def kernel(x, edge_index, W1, b1, W2, b2, W3, b3, W4, b4):
  row = edge_index[0].astype(jnp.int32)
  col = edge_index[1].astype(jnp.int32)
  pad = EPAD - E
  row_p = jnp.concatenate([row, jnp.full((pad,), N, jnp.int32)])
  col_p = jnp.concatenate([col, jnp.zeros((pad,), jnp.int32)])
  agg2 = _sc_agg(x, row_p, col_p, jnp.zeros((NP, D), jnp.float32))
  agg2 = agg2.reshape(NC, NP, D)

  agg = agg2[0, :N] + agg2[1, :N]
  deg = jnp.bincount(row, length=N).astype(x.dtype)
  deg = jnp.maximum(deg, 1.0)[:, None]
  agg = agg / deg
  h = jnp.concatenate([x, agg], axis=1)
  h = jax.nn.relu(h @ W1.T + b1)
  h = jax.nn.relu(h @ W2.T + b2)
  h = jax.nn.relu(h @ W3.T + b3)
  return h @ W4.T + b4
